# baseline (speedup 1.0000x reference)
import functools

import jax
import jax.numpy as jnp
from jax import lax
from jax.experimental import pallas as pl
from jax.experimental.pallas import tpu as pltpu
from jax.experimental.pallas import tpu_sc as plsc

NUM_ENTITIES = 1000000
NUM_RELATIONS = 1000
D = 64
B = 16384

NC = 2
NS = 16
L = 16
NW = NC * NS
BPW = B // NW
GCHUNK = 128

_mesh = plsc.VectorSubcoreMesh(core_axis_name="c", subcore_axis_name="s")


@functools.partial(
    pl.kernel,
    mesh=_mesh,
    out_type=jax.ShapeDtypeStruct((B,), jnp.float32),
    compiler_params=pltpu.CompilerParams(needs_layout_passes=False,
                                         use_tc_tiling_on_sc=False),
    scratch_types=[
        pltpu.VMEM((BPW,), jnp.int32),
        pltpu.VMEM((BPW,), jnp.int32),
        pltpu.VMEM((BPW,), jnp.int32),
        pltpu.VMEM((BPW, D), jnp.float32),
        pltpu.VMEM((BPW, D), jnp.float32),
        pltpu.VMEM((BPW, D), jnp.float32),
        pltpu.VMEM((BPW,), jnp.float32),
        pltpu.VMEM((L * BPW,), jnp.float32),
        pltpu.SemaphoreType.DMA,
    ],
)
def _distmult_sc(node_hbm, edge_hbm, e_hbm, p_hbm, u_hbm, out_hbm,
                 e_idx, p_idx, u_idx, e_rows, p_rows, u_rows, out_v, trans, sem):
    wid = lax.axis_index("s") * NC + lax.axis_index("c")
    base = wid * BPW

    pltpu.sync_copy(e_hbm.at[pl.ds(base, BPW)], e_idx)
    pltpu.sync_copy(p_hbm.at[pl.ds(base, BPW)], p_idx)
    pltpu.sync_copy(u_hbm.at[pl.ds(base, BPW)], u_idx)

    copies = []
    for j in range(BPW // GCHUNK):
        sl = pl.ds(j * GCHUNK, GCHUNK)
        copies.append(pltpu.async_copy(node_hbm.at[e_idx.at[sl]], e_rows.at[sl], sem))
        copies.append(pltpu.async_copy(node_hbm.at[u_idx.at[sl]], u_rows.at[sl], sem))
        copies.append(pltpu.async_copy(edge_hbm.at[p_idx.at[sl]], p_rows.at[sl], sem))
    for c in copies:
        c.wait()

    lane_off = lax.iota(jnp.int32, L) * BPW

    def elem(b, carry):
        acc = jnp.zeros((L,), jnp.float32)
        for c in range(D // L):
            sl = pl.ds(c * L, L)
            acc = acc + e_rows[b, sl] * p_rows[b, sl] * u_rows[b, sl]
        plsc.store_scatter(trans, [lane_off + b], acc)
        return carry

    lax.fori_loop(0, BPW, elem, 0, unroll=4)

    def red(g, carry):
        col0 = g * L
        s = trans[pl.ds(col0, L)]
        for j in range(1, L):
            s = s + trans[pl.ds(j * BPW + col0, L)]
        out_v[pl.ds(col0, L)] = s
        return carry

    lax.fori_loop(0, BPW // L, red, 0, unroll=2)

    pltpu.sync_copy(out_v, out_hbm.at[pl.ds(base, BPW)])


def kernel(node_embeddings, edge_embeddings, e_idc, p_idc, u_idc):
    return _distmult_sc(node_embeddings, edge_embeddings,
                        e_idc.astype(jnp.int32), p_idc.astype(jnp.int32),
                        u_idc.astype(jnp.int32))


# speedup vs baseline: 1.1021x; 1.1021x over previous
import functools

import jax
import jax.numpy as jnp
from jax import lax
from jax.experimental import pallas as pl
from jax.experimental.pallas import tpu as pltpu
from jax.experimental.pallas import tpu_sc as plsc

NUM_ENTITIES = 1000000
NUM_RELATIONS = 1000
D = 64
DP = 128
B = 16384

NC = 2
NS = 16
L = 16
NW = NC * NS
BPW = B // NW
WAVE = 256
GCHUNK = 128

_mesh = plsc.VectorSubcoreMesh(core_axis_name="c", subcore_axis_name="s")


@functools.partial(
    pl.kernel,
    mesh=_mesh,
    out_type=jax.ShapeDtypeStruct((B,), jnp.float32),
    compiler_params=pltpu.CompilerParams(needs_layout_passes=False),
    scratch_types=[
        pltpu.VMEM((BPW,), jnp.int32),
        pltpu.VMEM((BPW,), jnp.int32),
        pltpu.VMEM((BPW,), jnp.int32),
        pltpu.VMEM((WAVE, DP), jnp.float32),
        pltpu.VMEM((WAVE, DP), jnp.float32),
        pltpu.VMEM((WAVE, DP), jnp.float32),
        pltpu.VMEM((BPW,), jnp.float32),
        pltpu.VMEM((L * BPW,), jnp.float32),
        pltpu.SemaphoreType.DMA,
    ],
)
def _distmult_sc(node_hbm, edge_hbm, e_hbm, p_hbm, u_hbm, out_hbm,
                 e_idx, p_idx, u_idx, e_rows, p_rows, u_rows, out_v, trans,
                 sem):
    wid = lax.axis_index("s") * NC + lax.axis_index("c")
    base = wid * BPW

    pltpu.sync_copy(e_hbm.at[pl.ds(base, BPW)], e_idx)
    pltpu.sync_copy(p_hbm.at[pl.ds(base, BPW)], p_idx)
    pltpu.sync_copy(u_hbm.at[pl.ds(base, BPW)], u_idx)

    lane_off = lax.iota(jnp.int32, L) * BPW

    for wave in range(BPW // WAVE):
        w0 = wave * WAVE
        copies = []
        for j in range(WAVE // GCHUNK):
            isl = pl.ds(w0 + j * GCHUNK, GCHUNK)
            dsl = pl.ds(j * GCHUNK, GCHUNK)
            copies.append(pltpu.async_copy(
                node_hbm.at[e_idx.at[isl]], e_rows.at[dsl], sem))
            copies.append(pltpu.async_copy(
                node_hbm.at[u_idx.at[isl]], u_rows.at[dsl], sem))
            copies.append(pltpu.async_copy(
                edge_hbm.at[p_idx.at[isl]], p_rows.at[dsl], sem))
        for c in copies:
            c.wait()

        def elem(b, carry):
            acc = jnp.zeros((L,), jnp.float32)
            for c in range(D // L):
                sl = pl.ds(c * L, L)
                acc = acc + e_rows[b, sl] * p_rows[b, sl] * u_rows[b, sl]
            plsc.store_scatter(trans, [lane_off + (w0 + b)], acc)
            return carry

        lax.fori_loop(0, WAVE, elem, 0, unroll=4)

    def red(g, carry):
        col0 = g * L
        s = trans[pl.ds(col0, L)]
        for j in range(1, L):
            s = s + trans[pl.ds(j * BPW + col0, L)]
        out_v[pl.ds(col0, L)] = s
        return carry

    lax.fori_loop(0, BPW // L, red, 0, unroll=2)

    pltpu.sync_copy(out_v, out_hbm.at[pl.ds(base, BPW)])


def kernel(node_embeddings, edge_embeddings, e_idc, p_idc, u_idc):
    node128 = jnp.pad(node_embeddings, ((0, 0), (0, DP - D)))
    edge128 = jnp.pad(edge_embeddings, ((0, 0), (0, DP - D)))
    return _distmult_sc(node128, edge128,
                        e_idc.astype(jnp.int32), p_idc.astype(jnp.int32),
                        u_idc.astype(jnp.int32))
